# Initial kernel scaffold; baseline (speedup 1.0000x reference)
import jax
import jax.numpy as jnp
from jax.experimental import pallas as pl


def kernel(prefix, embedding_weight):
    raise NotImplementedError("write your pallas kernel here")


# speedup vs baseline: 1.6820x; 1.6820x over previous
import functools

import jax
import jax.numpy as jnp
from jax import lax
from jax.experimental import pallas as pl
from jax.experimental.pallas import tpu as pltpu
from jax.experimental.pallas import tpu_sc as plsc

_B = 8192
_D = 4096
_NW = 32
_BPW = _B // _NW
_R = 8
_NCH = _BPW // _R
_NBUF = 2


def _gather_kernel(idx_hbm, table_hbm, out_hbm, idx_v, bufs, gsems, wsems):
    wid = lax.axis_index("s") * 2 + lax.axis_index("c")
    base = wid * _BPW
    pltpu.sync_copy(idx_hbm.at[pl.ds(base, _BPW)], idx_v)

    def body(i, carry):
        for b in range(_NBUF):
            g = i * _NBUF + b

            @pl.when(i > 0)
            def _wait_prev_write():
                pltpu.make_async_copy(
                    bufs.at[b],
                    out_hbm.at[pl.ds(base + (g - _NBUF) * _R, _R)],
                    wsems.at[b]).wait()

            pltpu.async_copy(
                table_hbm.at[idx_v.at[pl.ds(g * _R, _R)]],
                bufs.at[b], gsems.at[b])
        for b in range(_NBUF):
            g = i * _NBUF + b
            pltpu.make_async_copy(
                table_hbm.at[idx_v.at[pl.ds(g * _R, _R)]],
                bufs.at[b], gsems.at[b]).wait()
            pltpu.async_copy(
                bufs.at[b], out_hbm.at[pl.ds(base + g * _R, _R)], wsems.at[b])
        return carry

    lax.fori_loop(0, _NCH // _NBUF, body, 0)

    for b in range(_NBUF):
        g = _NCH - _NBUF + b
        pltpu.make_async_copy(
            bufs.at[b], out_hbm.at[pl.ds(base + g * _R, _R)],
            wsems.at[b]).wait()


def kernel(prefix, embedding_weight):
    idx_flat = prefix.reshape(_B)
    out = functools.partial(
        pl.kernel,
        mesh=plsc.VectorSubcoreMesh(core_axis_name="c", subcore_axis_name="s"),
        out_type=jax.ShapeDtypeStruct((_B, _D), jnp.float32),
        scratch_types=[
            pltpu.VMEM((_BPW,), jnp.int32),
            pltpu.VMEM((_NBUF, _R, _D), jnp.float32),
            pltpu.SemaphoreType.DMA((_NBUF,)),
            pltpu.SemaphoreType.DMA((_NBUF,)),
        ],
    )(_gather_kernel)(idx_flat, embedding_weight)
    return out.reshape(4, 2048, _D)
